# Initial kernel scaffold; baseline (speedup 1.0000x reference)
import functools

import jax
import jax.numpy as jnp
from jax import lax
from jax.experimental import pallas as pl
from jax.experimental.pallas import tpu as pltpu
from jax.experimental.pallas import tpu_sc as plsc

N = 10000
NP = 10240
E = 320000
NW = 32
EPW = 10240
EP = NW * EPW
K = 128
CH = EPW // K
SUB = NP // 16
BM = 1024


def _make_sc_agg(width):
    mesh = plsc.VectorSubcoreMesh(core_axis_name="c", subcore_axis_name="s")

    @functools.partial(
        pl.kernel,
        out_type=jax.ShapeDtypeStruct((2, NP, width), jnp.float32),
        mesh=mesh,
        scratch_types=[
            pltpu.VMEM((K,), jnp.int32),
            pltpu.VMEM((K,), jnp.int32),
            pltpu.VMEM((K, width), jnp.float32),
            pltpu.VMEM_SHARED((NP, width), jnp.float32),
        ],
    )
    def agg(src_hbm, dst_hbm, tab_hbm, zero_hbm, out_hbm, idx_g, idx_s, rows, acc):
        cid = lax.axis_index("c")
        sid = lax.axis_index("s")
        w = cid * 16 + sid
        pltpu.sync_copy(zero_hbm.at[pl.ds(sid * SUB, SUB)],
                        acc.at[pl.ds(sid * SUB, SUB)])
        plsc.subcore_barrier()

        @pl.loop(0, CH)
        def _(c):
            base = w * EPW + c * K
            pltpu.sync_copy(src_hbm.at[pl.ds(base, K)], idx_g)
            pltpu.sync_copy(dst_hbm.at[pl.ds(base, K)], idx_s)
            pltpu.sync_copy(tab_hbm.at[idx_g], rows)
            pltpu.sync_copy(rows, acc.at[idx_s], add=True)

        plsc.subcore_barrier()
        pltpu.sync_copy(acc.at[pl.ds(sid * SUB, SUB)],
                        out_hbm.at[cid, pl.ds(sid * SUB, SUB)])

    return agg


_sc_agg128 = _make_sc_agg(128)
_sc_agg16 = _make_sc_agg(16)


def _dinv_block(dp):
    deg = dp[0, :, 0] + dp[1, :, 0] + 1.0
    return lax.rsqrt(deg)


def _tc_matmul(xp, W1):
    def body(x_ref, w_ref, o_ref):
        o_ref[...] = jnp.dot(x_ref[...], w_ref[...],
                             preferred_element_type=jnp.float32)

    return pl.pallas_call(
        body,
        grid=(NP // BM,),
        in_specs=[pl.BlockSpec((BM, 128), lambda i: (i, 0)),
                  pl.BlockSpec((128, 128), lambda i: (0, 0))],
        out_specs=pl.BlockSpec((BM, 128), lambda i: (i, 0)),
        out_shape=jax.ShapeDtypeStruct((NP, 128), jnp.float32),
    )(xp, W1)


def _tc_scale(deg_parts, h):
    def body(dp_ref, h_ref, o_ref):
        dinv = _dinv_block(dp_ref[...])
        o_ref[...] = dinv[:, None] * h_ref[...]

    return pl.pallas_call(
        body,
        grid=(NP // BM,),
        in_specs=[pl.BlockSpec((2, BM, 16), lambda i: (0, i, 0)),
                  pl.BlockSpec((BM, 128), lambda i: (i, 0))],
        out_specs=pl.BlockSpec((BM, 128), lambda i: (i, 0)),
        out_shape=jax.ShapeDtypeStruct((NP, 128), jnp.float32),
    )(deg_parts, h)


def _tc_layer(S, deg_parts, h, b1r, W2p, Wlp):
    def body(s_ref, dp_ref, h_ref, b1_ref, w2_ref, wl_ref, o_ref):
        dinv = _dinv_block(dp_ref[...])
        s = s_ref[0] + s_ref[1]
        h1 = dinv[:, None] * s + (dinv * dinv)[:, None] * h_ref[...] + b1_ref[...]
        r = jnp.maximum(h1, 0.0)
        w2l = jnp.dot(w2_ref[...], wl_ref[...],
                      preferred_element_type=jnp.float32)
        z = jnp.dot(r, w2l, preferred_element_type=jnp.float32)
        zg = dinv * z[:, 0]
        o_ref[...] = jnp.broadcast_to(zg[:, None], (BM, 16))

    return pl.pallas_call(
        body,
        grid=(NP // BM,),
        in_specs=[pl.BlockSpec((2, BM, 128), lambda i: (0, i, 0)),
                  pl.BlockSpec((2, BM, 16), lambda i: (0, i, 0)),
                  pl.BlockSpec((BM, 128), lambda i: (i, 0)),
                  pl.BlockSpec((1, 128), lambda i: (0, 0)),
                  pl.BlockSpec((128, 128), lambda i: (0, 0)),
                  pl.BlockSpec((128, 8), lambda i: (0, 0))],
        out_specs=pl.BlockSpec((BM, 16), lambda i: (i, 0)),
        out_shape=jax.ShapeDtypeStruct((NP, 16), jnp.float32),
    )(S, deg_parts, h, b1r, W2p, Wlp)


def _tc_final(T, zg16, deg_parts, b2p, Wlp, blp):
    def body(t_ref, zg_ref, dp_ref, b2_ref, wl_ref, bl_ref, o_ref):
        dinv = _dinv_block(dp_ref[...])
        t = t_ref[0, :, 0] + t_ref[1, :, 0]
        c2 = jnp.dot(b2_ref[...], wl_ref[...],
                     preferred_element_type=jnp.float32)[0, 0] + bl_ref[0, 0]
        out = dinv * t + dinv * zg_ref[:, 0] + c2
        o_ref[...] = jnp.broadcast_to(out[:, None], (BM, 8))

    return pl.pallas_call(
        body,
        grid=(NP // BM,),
        in_specs=[pl.BlockSpec((2, BM, 16), lambda i: (0, i, 0)),
                  pl.BlockSpec((BM, 16), lambda i: (i, 0)),
                  pl.BlockSpec((2, BM, 16), lambda i: (0, i, 0)),
                  pl.BlockSpec((1, 128), lambda i: (0, 0)),
                  pl.BlockSpec((128, 8), lambda i: (0, 0)),
                  pl.BlockSpec((1, 8), lambda i: (0, 0))],
        out_specs=pl.BlockSpec((BM, 8), lambda i: (i, 0)),
        out_shape=jax.ShapeDtypeStruct((NP, 8), jnp.float32),
    )(T, zg16, deg_parts, b2p, Wlp, blp)


def kernel(x, edge_index, W1, b1, W2, b2, Wl, bl):
    src = edge_index[0].astype(jnp.int32)
    dst = edge_index[1].astype(jnp.int32)
    padi = jnp.full((EP - E,), N, dtype=jnp.int32)
    srcp = jnp.concatenate([src, padi])
    dstp = jnp.concatenate([dst, padi])

    xp = jnp.zeros((NP, 128), jnp.float32).at[:N].set(x)
    zero128 = jnp.zeros((NP, 128), jnp.float32)
    zero16 = jnp.zeros((NP, 16), jnp.float32)
    ones16 = jnp.ones((NP, 16), jnp.float32)

    b1r = b1.reshape(1, 128)
    W2p = jnp.zeros((128, 128), jnp.float32).at[:, :100].set(W2)
    Wlp = jnp.zeros((128, 8), jnp.float32).at[:100, :1].set(Wl)
    b2p = jnp.zeros((1, 128), jnp.float32).at[0, :100].set(b2)
    blp = jnp.zeros((1, 8), jnp.float32).at[0, 0].set(bl[0])

    deg_parts = _sc_agg16(srcp, dstp, ones16, zero16)
    h = _tc_matmul(xp, W1)

    g = _tc_scale(deg_parts, h)
    S = _sc_agg128(srcp, dstp, g, zero128)
    zg16 = _tc_layer(S, deg_parts, h, b1r, W2p, Wlp)
    T = _sc_agg16(srcp, dstp, zg16, zero16)
    outp = _tc_final(T, zg16, deg_parts, b2p, Wlp, blp)
    return outp[:N, :1]


# speedup vs baseline: 9.7129x; 9.7129x over previous
import functools

import jax
import jax.numpy as jnp
from jax import lax
from jax.experimental import pallas as pl
from jax.experimental.pallas import tpu as pltpu
from jax.experimental.pallas import tpu_sc as plsc

N = 10000
NP = 10240
E = 320000
NW = 32
EPW = 10240
EP = NW * EPW
K = 128
CH = EPW // K
SUB = NP // 16
BM = 1024


def _make_sc_agg(width):
    mesh = plsc.VectorSubcoreMesh(core_axis_name="c", subcore_axis_name="s")

    @functools.partial(
        pl.kernel,
        out_type=jax.ShapeDtypeStruct((2, NP, width), jnp.float32),
        mesh=mesh,
        scratch_types=[
            pltpu.VMEM((K,), jnp.int32),
            pltpu.VMEM((K,), jnp.int32),
            pltpu.VMEM((K, width), jnp.float32),
            pltpu.VMEM_SHARED((NP, width), jnp.float32),
        ],
        compiler_params=pltpu.CompilerParams(use_tc_tiling_on_sc=False),
    )
    def agg(src_hbm, dst_hbm, tab_hbm, zero_hbm, out_hbm, idx_g, idx_s, rows, acc):
        cid = lax.axis_index("c")
        sid = lax.axis_index("s")
        w = cid * 16 + sid
        pltpu.sync_copy(zero_hbm.at[pl.ds(sid * SUB, SUB)],
                        acc.at[pl.ds(sid * SUB, SUB)])
        plsc.subcore_barrier()

        @pl.loop(0, CH)
        def _(c):
            base = w * EPW + c * K
            pltpu.sync_copy(src_hbm.at[pl.ds(base, K)], idx_g)
            pltpu.sync_copy(dst_hbm.at[pl.ds(base, K)], idx_s)
            pltpu.sync_copy(tab_hbm.at[idx_g], rows)
            pltpu.sync_copy(rows, acc.at[idx_s], add=True)

        plsc.subcore_barrier()
        pltpu.sync_copy(acc.at[pl.ds(sid * SUB, SUB)],
                        out_hbm.at[cid, pl.ds(sid * SUB, SUB)])

    return agg


_sc_agg128 = _make_sc_agg(128)
_sc_agg16 = _make_sc_agg(16)


def _dinv_block(dp):
    deg = dp[0, :, 0] + dp[1, :, 0] + 1.0
    return lax.rsqrt(deg)


def _tc_matmul(xp, W1):
    def body(x_ref, w_ref, o_ref):
        o_ref[...] = jnp.dot(x_ref[...], w_ref[...],
                             preferred_element_type=jnp.float32)

    return pl.pallas_call(
        body,
        grid=(NP // BM,),
        in_specs=[pl.BlockSpec((BM, 128), lambda i: (i, 0)),
                  pl.BlockSpec((128, 128), lambda i: (0, 0))],
        out_specs=pl.BlockSpec((BM, 128), lambda i: (i, 0)),
        out_shape=jax.ShapeDtypeStruct((NP, 128), jnp.float32),
    )(xp, W1)


def _tc_scale(deg_parts, h):
    def body(dp_ref, h_ref, o_ref):
        dinv = _dinv_block(dp_ref[...])
        o_ref[...] = dinv[:, None] * h_ref[...]

    return pl.pallas_call(
        body,
        grid=(NP // BM,),
        in_specs=[pl.BlockSpec((2, BM, 16), lambda i: (0, i, 0)),
                  pl.BlockSpec((BM, 128), lambda i: (i, 0))],
        out_specs=pl.BlockSpec((BM, 128), lambda i: (i, 0)),
        out_shape=jax.ShapeDtypeStruct((NP, 128), jnp.float32),
    )(deg_parts, h)


def _tc_layer(S, deg_parts, h, b1r, W2p, Wlp):
    def body(s_ref, dp_ref, h_ref, b1_ref, w2_ref, wl_ref, o_ref):
        dinv = _dinv_block(dp_ref[...])
        s = s_ref[0] + s_ref[1]
        h1 = dinv[:, None] * s + (dinv * dinv)[:, None] * h_ref[...] + b1_ref[...]
        r = jnp.maximum(h1, 0.0)
        w2l = jnp.dot(w2_ref[...], wl_ref[...],
                      preferred_element_type=jnp.float32)
        z = jnp.dot(r, w2l, preferred_element_type=jnp.float32)
        zg = dinv * z[:, 0]
        o_ref[...] = jnp.broadcast_to(zg[:, None], (BM, 16))

    return pl.pallas_call(
        body,
        grid=(NP // BM,),
        in_specs=[pl.BlockSpec((2, BM, 128), lambda i: (0, i, 0)),
                  pl.BlockSpec((2, BM, 16), lambda i: (0, i, 0)),
                  pl.BlockSpec((BM, 128), lambda i: (i, 0)),
                  pl.BlockSpec((1, 128), lambda i: (0, 0)),
                  pl.BlockSpec((128, 128), lambda i: (0, 0)),
                  pl.BlockSpec((128, 8), lambda i: (0, 0))],
        out_specs=pl.BlockSpec((BM, 16), lambda i: (i, 0)),
        out_shape=jax.ShapeDtypeStruct((NP, 16), jnp.float32),
    )(S, deg_parts, h, b1r, W2p, Wlp)


def _tc_final(T, zg16, deg_parts, b2p, Wlp, blp):
    def body(t_ref, zg_ref, dp_ref, b2_ref, wl_ref, bl_ref, o_ref):
        dinv = _dinv_block(dp_ref[...])
        t = t_ref[0, :, 0] + t_ref[1, :, 0]
        c2 = jnp.dot(b2_ref[...], wl_ref[...],
                     preferred_element_type=jnp.float32)[0, 0] + bl_ref[0, 0]
        out = dinv * t + dinv * zg_ref[:, 0] + c2
        o_ref[...] = jnp.broadcast_to(out[:, None], (BM, 8))

    return pl.pallas_call(
        body,
        grid=(NP // BM,),
        in_specs=[pl.BlockSpec((2, BM, 16), lambda i: (0, i, 0)),
                  pl.BlockSpec((BM, 16), lambda i: (i, 0)),
                  pl.BlockSpec((2, BM, 16), lambda i: (0, i, 0)),
                  pl.BlockSpec((1, 128), lambda i: (0, 0)),
                  pl.BlockSpec((128, 8), lambda i: (0, 0)),
                  pl.BlockSpec((1, 8), lambda i: (0, 0))],
        out_specs=pl.BlockSpec((BM, 8), lambda i: (i, 0)),
        out_shape=jax.ShapeDtypeStruct((NP, 8), jnp.float32),
    )(T, zg16, deg_parts, b2p, Wlp, blp)


def kernel(x, edge_index, W1, b1, W2, b2, Wl, bl):
    src = edge_index[0].astype(jnp.int32)
    dst = edge_index[1].astype(jnp.int32)
    padi = jnp.full((EP - E,), N, dtype=jnp.int32)
    srcp = jnp.concatenate([src, padi])
    dstp = jnp.concatenate([dst, padi])

    xp = jnp.zeros((NP, 128), jnp.float32).at[:N].set(x)
    zero128 = jnp.zeros((NP, 128), jnp.float32)
    zero16 = jnp.zeros((NP, 16), jnp.float32)
    ones16 = jnp.ones((NP, 16), jnp.float32)

    b1r = b1.reshape(1, 128)
    W2p = jnp.zeros((128, 128), jnp.float32).at[:, :100].set(W2)
    Wlp = jnp.zeros((128, 8), jnp.float32).at[:100, :1].set(Wl)
    b2p = jnp.zeros((1, 128), jnp.float32).at[0, :100].set(b2)
    blp = jnp.zeros((1, 8), jnp.float32).at[0, 0].set(bl[0])

    deg_parts = _sc_agg16(srcp, dstp, ones16, zero16)
    h = _tc_matmul(xp, W1)

    g = _tc_scale(deg_parts, h)
    S = _sc_agg128(srcp, dstp, g, zero128)
    zg16 = _tc_layer(S, deg_parts, h, b1r, W2p, Wlp)
    T = _sc_agg16(srcp, dstp, zg16, zero16)
    outp = _tc_final(T, zg16, deg_parts, b2p, Wlp, blp)
    return outp[:N, :1]


# speedup vs baseline: 16.1204x; 1.6597x over previous
import functools

import jax
import jax.numpy as jnp
from jax import lax
from jax.experimental import pallas as pl
from jax.experimental.pallas import tpu as pltpu
from jax.experimental.pallas import tpu_sc as plsc

N = 10000
NP = 10240
E = 320000
NW = 32
EW = 10000
K = 80
CH = 128
CHP = CH + 4
U = 4
BATCH = CH // U
SUB = NP // 16
BM = 1024


def _make_sc_agg(width):
    mesh = plsc.VectorSubcoreMesh(core_axis_name="c", subcore_axis_name="s")

    @functools.partial(
        pl.kernel,
        out_type=jax.ShapeDtypeStruct((2, NP, width), jnp.float32),
        mesh=mesh,
        scratch_types=[
            pltpu.VMEM((2, U, K), jnp.int32),
            pltpu.VMEM((2, U, K), jnp.int32),
            pltpu.VMEM((U, K, width), jnp.float32),
            pltpu.VMEM_SHARED((NP, width), jnp.float32),
            pltpu.SemaphoreType.DMA((2 * U,)),
            pltpu.SemaphoreType.DMA((2 * U,)),
            pltpu.SemaphoreType.DMA((U,)),
            pltpu.SemaphoreType.DMA((U,)),
        ],
        compiler_params=pltpu.CompilerParams(use_tc_tiling_on_sc=False),
    )
    def agg(src_hbm, dst_hbm, tab_hbm, zero_hbm, out_hbm,
            idxg, idxs, rows, acc, si_g, si_s, sg, ss):
        cid = lax.axis_index("c")
        sid = lax.axis_index("s")
        w = cid * 16 + sid
        pltpu.sync_copy(zero_hbm.at[pl.ds(sid * SUB, SUB)],
                        acc.at[pl.ds(sid * SUB, SUB)])
        for j in range(U):
            pltpu.sync_copy(src_hbm.at[w, j], idxg.at[0, j])
            pltpu.sync_copy(dst_hbm.at[w, j], idxs.at[0, j])
        plsc.subcore_barrier()

        def run_batch(t, bank, wait_idx, prefetch):
            other = 1 - bank
            for j in range(U):
                if wait_idx:
                    pltpu.make_async_copy(src_hbm.at[w, 0], idxg.at[bank, j],
                                          si_g.at[bank * U + j]).wait()
                pltpu.async_copy(tab_hbm.at[idxg.at[bank, j]], rows.at[j],
                                 sg.at[j])
            if prefetch:
                for j in range(U):
                    c = (t + 1) * U + j
                    pltpu.async_copy(src_hbm.at[w, c], idxg.at[other, j],
                                     si_g.at[other * U + j])
                    pltpu.async_copy(dst_hbm.at[w, c], idxs.at[other, j],
                                     si_s.at[other * U + j])
            for j in range(U):
                if wait_idx:
                    pltpu.make_async_copy(dst_hbm.at[w, 0], idxs.at[bank, j],
                                          si_s.at[bank * U + j]).wait()
                pltpu.make_async_copy(tab_hbm.at[idxg.at[bank, j]], rows.at[j],
                                      sg.at[j]).wait()
                pltpu.async_copy(rows.at[j], acc.at[idxs.at[bank, j]],
                                 ss.at[j], add=True)
            for j in range(U):
                pltpu.make_async_copy(rows.at[j], acc.at[idxs.at[bank, j]],
                                      ss.at[j]).wait()

        run_batch(0, 0, wait_idx=False, prefetch=True)

        @pl.loop(0, (BATCH - 2) // 2)
        def _(tt):
            run_batch(2 * tt + 1, 1, wait_idx=True, prefetch=True)
            run_batch(2 * tt + 2, 0, wait_idx=True, prefetch=True)

        run_batch(BATCH - 1, 1, wait_idx=True, prefetch=False)

        plsc.subcore_barrier()
        pltpu.sync_copy(acc.at[pl.ds(sid * SUB, SUB)],
                        out_hbm.at[cid, pl.ds(sid * SUB, SUB)])

    return agg


def _make_sc_deg():
    mesh = plsc.VectorSubcoreMesh(core_axis_name="c", subcore_axis_name="s")

    @functools.partial(
        pl.kernel,
        out_type=jax.ShapeDtypeStruct((2, NP, 16), jnp.float32),
        mesh=mesh,
        scratch_types=[
            pltpu.VMEM((CH, K), jnp.int32),
            pltpu.VMEM((K, 16), jnp.float32),
            pltpu.VMEM_SHARED((NP, 16), jnp.float32),
            pltpu.SemaphoreType.DMA((U,)),
        ],
        compiler_params=pltpu.CompilerParams(use_tc_tiling_on_sc=False),
    )
    def deg(dst_hbm, ones_hbm, zero_hbm, out_hbm, idx_s, vals, acc, ss):
        cid = lax.axis_index("c")
        sid = lax.axis_index("s")
        w = cid * 16 + sid
        pltpu.sync_copy(dst_hbm.at[w, pl.ds(0, CH)], idx_s)
        pltpu.sync_copy(ones_hbm, vals)
        pltpu.sync_copy(zero_hbm.at[pl.ds(sid * SUB, SUB)],
                        acc.at[pl.ds(sid * SUB, SUB)])
        plsc.subcore_barrier()

        @pl.loop(0, BATCH)
        def _(it):
            c0 = it * U
            sds = [pltpu.async_copy(vals, acc.at[idx_s.at[c0 + j]],
                                    ss.at[j], add=True)
                   for j in range(U)]
            for j in range(U):
                sds[j].wait()

        plsc.subcore_barrier()
        pltpu.sync_copy(acc.at[pl.ds(sid * SUB, SUB)],
                        out_hbm.at[cid, pl.ds(sid * SUB, SUB)])

    return deg


_sc_agg128 = _make_sc_agg(128)
_sc_agg16 = _make_sc_agg(16)
_sc_deg = _make_sc_deg()


def _dinv_block(dp):
    deg = dp[0, :, 0] + dp[1, :, 0] + 1.0
    return lax.rsqrt(deg)


def _tc_matmul(xp, W1):
    def body(x_ref, w_ref, o_ref):
        o_ref[...] = jnp.dot(x_ref[...], w_ref[...],
                             preferred_element_type=jnp.float32)

    return pl.pallas_call(
        body,
        grid=(NP // BM,),
        in_specs=[pl.BlockSpec((BM, 128), lambda i: (i, 0)),
                  pl.BlockSpec((128, 128), lambda i: (0, 0))],
        out_specs=pl.BlockSpec((BM, 128), lambda i: (i, 0)),
        out_shape=jax.ShapeDtypeStruct((NP, 128), jnp.float32),
    )(xp, W1)


def _tc_scale(deg_parts, h):
    def body(dp_ref, h_ref, o_ref):
        dinv = _dinv_block(dp_ref[...])
        o_ref[...] = dinv[:, None] * h_ref[...]

    return pl.pallas_call(
        body,
        grid=(NP // BM,),
        in_specs=[pl.BlockSpec((2, BM, 16), lambda i: (0, i, 0)),
                  pl.BlockSpec((BM, 128), lambda i: (i, 0))],
        out_specs=pl.BlockSpec((BM, 128), lambda i: (i, 0)),
        out_shape=jax.ShapeDtypeStruct((NP, 128), jnp.float32),
    )(deg_parts, h)


def _tc_layer(S, deg_parts, h, b1r, W2p, Wlp):
    def body(s_ref, dp_ref, h_ref, b1_ref, w2_ref, wl_ref, o_ref):
        dinv = _dinv_block(dp_ref[...])
        s = s_ref[0] + s_ref[1]
        h1 = dinv[:, None] * s + (dinv * dinv)[:, None] * h_ref[...] + b1_ref[...]
        r = jnp.maximum(h1, 0.0)
        w2l = jnp.dot(w2_ref[...], wl_ref[...],
                      preferred_element_type=jnp.float32)
        z = jnp.dot(r, w2l, preferred_element_type=jnp.float32)
        zg = dinv * z[:, 0]
        o_ref[...] = jnp.broadcast_to(zg[:, None], (BM, 16))

    return pl.pallas_call(
        body,
        grid=(NP // BM,),
        in_specs=[pl.BlockSpec((2, BM, 128), lambda i: (0, i, 0)),
                  pl.BlockSpec((2, BM, 16), lambda i: (0, i, 0)),
                  pl.BlockSpec((BM, 128), lambda i: (i, 0)),
                  pl.BlockSpec((1, 128), lambda i: (0, 0)),
                  pl.BlockSpec((128, 128), lambda i: (0, 0)),
                  pl.BlockSpec((128, 8), lambda i: (0, 0))],
        out_specs=pl.BlockSpec((BM, 16), lambda i: (i, 0)),
        out_shape=jax.ShapeDtypeStruct((NP, 16), jnp.float32),
    )(S, deg_parts, h, b1r, W2p, Wlp)


def _tc_final(T, zg16, deg_parts, b2p, Wlp, blp):
    def body(t_ref, zg_ref, dp_ref, b2_ref, wl_ref, bl_ref, o_ref):
        dinv = _dinv_block(dp_ref[...])
        t = t_ref[0, :, 0] + t_ref[1, :, 0]
        c2 = jnp.dot(b2_ref[...], wl_ref[...],
                     preferred_element_type=jnp.float32)[0, 0] + bl_ref[0, 0]
        out = dinv * t + dinv * zg_ref[:, 0] + c2
        o_ref[...] = jnp.broadcast_to(out[:, None], (BM, 8))

    return pl.pallas_call(
        body,
        grid=(NP // BM,),
        in_specs=[pl.BlockSpec((2, BM, 16), lambda i: (0, i, 0)),
                  pl.BlockSpec((BM, 16), lambda i: (i, 0)),
                  pl.BlockSpec((2, BM, 16), lambda i: (0, i, 0)),
                  pl.BlockSpec((1, 128), lambda i: (0, 0)),
                  pl.BlockSpec((128, 8), lambda i: (0, 0)),
                  pl.BlockSpec((1, 8), lambda i: (0, 0))],
        out_specs=pl.BlockSpec((BM, 8), lambda i: (i, 0)),
        out_shape=jax.ShapeDtypeStruct((NP, 8), jnp.float32),
    )(T, zg16, deg_parts, b2p, Wlp, blp)


def kernel(x, edge_index, W1, b1, W2, b2, Wl, bl):
    src = edge_index[0].astype(jnp.int32)
    dst = edge_index[1].astype(jnp.int32)
    srcp = jnp.pad(src.reshape(NW, EW), ((0, 0), (0, CHP * K - EW)),
                   constant_values=N).reshape(NW, CHP, K)
    dstp = jnp.pad(dst.reshape(NW, EW), ((0, 0), (0, CHP * K - EW)),
                   constant_values=N).reshape(NW, CHP, K)

    xp = jnp.zeros((NP, 128), jnp.float32).at[:N].set(x)
    zero128 = jnp.zeros((NP, 128), jnp.float32)
    zero16 = jnp.zeros((NP, 16), jnp.float32)
    ones16 = jnp.ones((K, 16), jnp.float32)

    b1r = b1.reshape(1, 128)
    W2p = jnp.zeros((128, 128), jnp.float32).at[:, :100].set(W2)
    Wlp = jnp.zeros((128, 8), jnp.float32).at[:100, :1].set(Wl)
    b2p = jnp.zeros((1, 128), jnp.float32).at[0, :100].set(b2)
    blp = jnp.zeros((1, 8), jnp.float32).at[0, 0].set(bl[0])

    deg_parts = _sc_deg(dstp, ones16, zero16)
    h = _tc_matmul(xp, W1)

    g = _tc_scale(deg_parts, h)
    S = _sc_agg128(srcp, dstp, g, zero128)
    zg16 = _tc_layer(S, deg_parts, h, b1r, W2p, Wlp)
    T = _sc_agg16(srcp, dstp, zg16, zero16)
    outp = _tc_final(T, zg16, deg_parts, b2p, Wlp, blp)
    return outp[:N, :1]


# speedup vs baseline: 25.1789x; 1.5619x over previous
import functools

import jax
import jax.numpy as jnp
from jax import lax
from jax.experimental import pallas as pl
from jax.experimental.pallas import tpu as pltpu
from jax.experimental.pallas import tpu_sc as plsc

N = 10000
NP = 10240
E = 320000
NW = 32
EW = 10000
K = 112
CH = 90
U = 3
CHP = CH + U
BATCH = CH // U
SUB = NP // 16
BM = 1024


def _make_sc_agg(width):
    mesh = plsc.VectorSubcoreMesh(core_axis_name="c", subcore_axis_name="s")

    @functools.partial(
        pl.kernel,
        out_type=jax.ShapeDtypeStruct((2, NP, width), jnp.float32),
        mesh=mesh,
        scratch_types=[
            pltpu.VMEM((2, U, K), jnp.int32),
            pltpu.VMEM((2, U, K), jnp.int32),
            pltpu.VMEM((U, K, width), jnp.float32),
            pltpu.VMEM_SHARED((NP, width), jnp.float32),
            pltpu.SemaphoreType.DMA((2 * U,)),
            pltpu.SemaphoreType.DMA((2 * U,)),
            pltpu.SemaphoreType.DMA((U,)),
            pltpu.SemaphoreType.DMA((U,)),
        ],
        compiler_params=pltpu.CompilerParams(use_tc_tiling_on_sc=False),
    )
    def agg(src_hbm, dst_hbm, tab_hbm, zero_hbm, out_hbm,
            idxg, idxs, rows, acc, si_g, si_s, sg, ss):
        cid = lax.axis_index("c")
        sid = lax.axis_index("s")
        w = cid * 16 + sid
        pltpu.sync_copy(zero_hbm.at[pl.ds(sid * SUB, SUB)],
                        acc.at[pl.ds(sid * SUB, SUB)])
        for j in range(U):
            pltpu.sync_copy(src_hbm.at[w, j], idxg.at[0, j])
            pltpu.sync_copy(dst_hbm.at[w, j], idxs.at[0, j])
        plsc.subcore_barrier()

        def run_batch(t, bank, wait_idx, prefetch):
            other = 1 - bank
            for j in range(U):
                if wait_idx:
                    pltpu.make_async_copy(src_hbm.at[w, 0], idxg.at[bank, j],
                                          si_g.at[bank * U + j]).wait()
                pltpu.async_copy(tab_hbm.at[idxg.at[bank, j]], rows.at[j],
                                 sg.at[j])
            if prefetch:
                for j in range(U):
                    c = (t + 1) * U + j
                    pltpu.async_copy(src_hbm.at[w, c], idxg.at[other, j],
                                     si_g.at[other * U + j])
                    pltpu.async_copy(dst_hbm.at[w, c], idxs.at[other, j],
                                     si_s.at[other * U + j])
            for j in range(U):
                if wait_idx:
                    pltpu.make_async_copy(dst_hbm.at[w, 0], idxs.at[bank, j],
                                          si_s.at[bank * U + j]).wait()
                pltpu.make_async_copy(tab_hbm.at[idxg.at[bank, j]], rows.at[j],
                                      sg.at[j]).wait()
                pltpu.async_copy(rows.at[j], acc.at[idxs.at[bank, j]],
                                 ss.at[j], add=True)
            for j in range(U):
                pltpu.make_async_copy(rows.at[j], acc.at[idxs.at[bank, j]],
                                      ss.at[j]).wait()

        run_batch(0, 0, wait_idx=False, prefetch=True)

        @pl.loop(0, (BATCH - 2) // 2)
        def _(tt):
            run_batch(2 * tt + 1, 1, wait_idx=True, prefetch=True)
            run_batch(2 * tt + 2, 0, wait_idx=True, prefetch=True)

        run_batch(BATCH - 1, 1, wait_idx=True, prefetch=False)

        plsc.subcore_barrier()
        pltpu.sync_copy(acc.at[pl.ds(sid * SUB, SUB)],
                        out_hbm.at[cid, pl.ds(sid * SUB, SUB)])

    return agg


def _make_sc_deg():
    mesh = plsc.VectorSubcoreMesh(core_axis_name="c", subcore_axis_name="s")

    @functools.partial(
        pl.kernel,
        out_type=jax.ShapeDtypeStruct((2, NP, 16), jnp.float32),
        mesh=mesh,
        scratch_types=[
            pltpu.VMEM((CH, K), jnp.int32),
            pltpu.VMEM((K, 16), jnp.float32),
            pltpu.VMEM_SHARED((NP, 16), jnp.float32),
            pltpu.SemaphoreType.DMA((U,)),
        ],
        compiler_params=pltpu.CompilerParams(use_tc_tiling_on_sc=False),
    )
    def deg(dst_hbm, ones_hbm, zero_hbm, out_hbm, idx_s, vals, acc, ss):
        cid = lax.axis_index("c")
        sid = lax.axis_index("s")
        w = cid * 16 + sid
        pltpu.sync_copy(dst_hbm.at[w, pl.ds(0, CH)], idx_s)
        pltpu.sync_copy(ones_hbm, vals)
        pltpu.sync_copy(zero_hbm.at[pl.ds(sid * SUB, SUB)],
                        acc.at[pl.ds(sid * SUB, SUB)])
        plsc.subcore_barrier()

        @pl.loop(0, BATCH)
        def _(it):
            c0 = it * U
            sds = [pltpu.async_copy(vals, acc.at[idx_s.at[c0 + j]],
                                    ss.at[j], add=True)
                   for j in range(U)]
            for j in range(U):
                sds[j].wait()

        plsc.subcore_barrier()
        pltpu.sync_copy(acc.at[pl.ds(sid * SUB, SUB)],
                        out_hbm.at[cid, pl.ds(sid * SUB, SUB)])

    return deg


_sc_agg128 = _make_sc_agg(128)
_sc_agg16 = _make_sc_agg(16)
_sc_deg = _make_sc_deg()


def _dinv_block(dp):
    deg = dp[0, :, 0] + dp[1, :, 0] + 1.0
    return lax.rsqrt(deg)


def _tc_matmul_scale(xp, W1, deg_parts):
    def body(x_ref, w_ref, dp_ref, h_ref, g_ref):
        h = jnp.dot(x_ref[...], w_ref[...], preferred_element_type=jnp.float32)
        dinv = _dinv_block(dp_ref[...])
        h_ref[...] = h
        g_ref[...] = dinv[:, None] * h

    return pl.pallas_call(
        body,
        grid=(NP // BM,),
        in_specs=[pl.BlockSpec((BM, 128), lambda i: (i, 0)),
                  pl.BlockSpec((128, 128), lambda i: (0, 0)),
                  pl.BlockSpec((2, BM, 16), lambda i: (0, i, 0))],
        out_specs=[pl.BlockSpec((BM, 128), lambda i: (i, 0)),
                   pl.BlockSpec((BM, 128), lambda i: (i, 0))],
        out_shape=[jax.ShapeDtypeStruct((NP, 128), jnp.float32),
                   jax.ShapeDtypeStruct((NP, 128), jnp.float32)],
    )(xp, W1, deg_parts)


def _tc_layer(S, deg_parts, h, b1r, W2p, Wlp):
    def body(s_ref, dp_ref, h_ref, b1_ref, w2_ref, wl_ref, o_ref):
        dinv = _dinv_block(dp_ref[...])
        s = s_ref[0] + s_ref[1]
        h1 = dinv[:, None] * s + (dinv * dinv)[:, None] * h_ref[...] + b1_ref[...]
        r = jnp.maximum(h1, 0.0)
        w2l = jnp.dot(w2_ref[...], wl_ref[...],
                      preferred_element_type=jnp.float32)
        z = jnp.dot(r, w2l, preferred_element_type=jnp.float32)
        zg = dinv * z[:, 0]
        o_ref[...] = jnp.broadcast_to(zg[:, None], (BM, 16))

    return pl.pallas_call(
        body,
        grid=(NP // BM,),
        in_specs=[pl.BlockSpec((2, BM, 128), lambda i: (0, i, 0)),
                  pl.BlockSpec((2, BM, 16), lambda i: (0, i, 0)),
                  pl.BlockSpec((BM, 128), lambda i: (i, 0)),
                  pl.BlockSpec((1, 128), lambda i: (0, 0)),
                  pl.BlockSpec((128, 128), lambda i: (0, 0)),
                  pl.BlockSpec((128, 8), lambda i: (0, 0))],
        out_specs=pl.BlockSpec((BM, 16), lambda i: (i, 0)),
        out_shape=jax.ShapeDtypeStruct((NP, 16), jnp.float32),
    )(S, deg_parts, h, b1r, W2p, Wlp)


def _tc_final(T, zg16, deg_parts, b2p, Wlp, blp):
    def body(t_ref, zg_ref, dp_ref, b2_ref, wl_ref, bl_ref, o_ref):
        dinv = _dinv_block(dp_ref[...])
        t = t_ref[0, :, 0] + t_ref[1, :, 0]
        c2 = jnp.dot(b2_ref[...], wl_ref[...],
                     preferred_element_type=jnp.float32)[0, 0] + bl_ref[0, 0]
        out = dinv * t + dinv * zg_ref[:, 0] + c2
        o_ref[...] = jnp.broadcast_to(out[:, None], (BM, 8))

    return pl.pallas_call(
        body,
        grid=(NP // BM,),
        in_specs=[pl.BlockSpec((2, BM, 16), lambda i: (0, i, 0)),
                  pl.BlockSpec((BM, 16), lambda i: (i, 0)),
                  pl.BlockSpec((2, BM, 16), lambda i: (0, i, 0)),
                  pl.BlockSpec((1, 128), lambda i: (0, 0)),
                  pl.BlockSpec((128, 8), lambda i: (0, 0)),
                  pl.BlockSpec((1, 8), lambda i: (0, 0))],
        out_specs=pl.BlockSpec((BM, 8), lambda i: (i, 0)),
        out_shape=jax.ShapeDtypeStruct((NP, 8), jnp.float32),
    )(T, zg16, deg_parts, b2p, Wlp, blp)


def kernel(x, edge_index, W1, b1, W2, b2, Wl, bl):
    src = edge_index[0].astype(jnp.int32)
    dst = edge_index[1].astype(jnp.int32)
    srcp = jnp.pad(src.reshape(NW, EW), ((0, 0), (0, CHP * K - EW)),
                   constant_values=N).reshape(NW, CHP, K)
    dstp = jnp.pad(dst.reshape(NW, EW), ((0, 0), (0, CHP * K - EW)),
                   constant_values=N).reshape(NW, CHP, K)

    xp = jnp.zeros((NP, 128), jnp.float32).at[:N].set(x)
    zero128 = jnp.zeros((NP, 128), jnp.float32)
    zero16 = jnp.zeros((NP, 16), jnp.float32)
    ones16 = jnp.ones((K, 16), jnp.float32)

    b1r = b1.reshape(1, 128)
    W2p = jnp.zeros((128, 128), jnp.float32).at[:, :100].set(W2)
    Wlp = jnp.zeros((128, 8), jnp.float32).at[:100, :1].set(Wl)
    b2p = jnp.zeros((1, 128), jnp.float32).at[0, :100].set(b2)
    blp = jnp.zeros((1, 8), jnp.float32).at[0, 0].set(bl[0])

    deg_parts = _sc_deg(dstp, ones16, zero16)
    h, g = _tc_matmul_scale(xp, W1, deg_parts)
    S = _sc_agg128(srcp, dstp, g, zero128)
    zg16 = _tc_layer(S, deg_parts, h, b1r, W2p, Wlp)
    T = _sc_agg16(srcp, dstp, zg16, zero16)
    outp = _tc_final(T, zg16, deg_parts, b2p, Wlp, blp)
    return outp[:N, :1]
